# Initial kernel scaffold; baseline (speedup 1.0000x reference)
import functools

import jax
import jax.numpy as jnp
from jax import lax
from jax.experimental import pallas as pl
from jax.experimental.pallas import tpu as pltpu
from jax.experimental.pallas import tpu_sc as plsc

N = 10000
E = 320000
D = 128
H1 = 16

NC = 2
NS = 16
NW = NC * NS
EW = E // NW
NCHUNK = 5
CHUNK = EW // NCHUNK
SB = 80
NSB = EW // SB
SBC = CHUNK // SB

_mesh = plsc.VectorSubcoreMesh(
    core_axis_name="c", subcore_axis_name="s", num_cores=NC, num_subcores=NS
)


def _leaky_exp(v):
    return jnp.exp(jnp.where(v >= 0.0, v, 0.2 * v))


def _tc1_body(x_ref, w_ref, v_ref, h_ref, oa_ref):
    xb = x_ref[...]
    h_ref[...] = jnp.dot(xb, w_ref[...], preferred_element_type=jnp.float32)
    oa_ref[...] = jnp.dot(xb, v_ref[...], preferred_element_type=jnp.float32)


def _tc1(x, w1s, vsd):
    r = 1000
    return pl.pallas_call(
        _tc1_body,
        grid=(N // r,),
        in_specs=[
            pl.BlockSpec((r, D), lambda i: (i, 0)),
            pl.BlockSpec((D, H1), lambda i: (0, 0)),
            pl.BlockSpec((D, 2), lambda i: (0, 0)),
        ],
        out_specs=[
            pl.BlockSpec((r, H1), lambda i: (i, 0)),
            pl.BlockSpec((r, 2), lambda i: (i, 0)),
        ],
        out_shape=[
            jax.ShapeDtypeStruct((N, H1), jnp.float32),
            jax.ShapeDtypeStruct((N, 2), jnp.float32),
        ],
    )(x, w1s, vsd)


def _tc2_body(op_ref, dp_ref, b1_ref, w2_ref, h2_ref):
    p = op_ref[0] + op_ref[1]
    d = dp_ref[0] + dp_ref[1]
    h1 = jnp.maximum(p / (d[:, None] + 1e-16) + b1_ref[...], 0.0)
    h2 = jnp.dot(h1, w2_ref[...], preferred_element_type=jnp.float32)
    h2_ref[...] = h2[:, 0]


def _tc2(op, dp, b1, w2):
    r = 1000
    return pl.pallas_call(
        _tc2_body,
        grid=(N // r,),
        in_specs=[
            pl.BlockSpec((NC, r, H1), lambda i: (0, i, 0)),
            pl.BlockSpec((NC, r), lambda i: (0, i)),
            pl.BlockSpec((1, H1), lambda i: (0, 0)),
            pl.BlockSpec((H1, 1), lambda i: (0, 0)),
        ],
        out_specs=pl.BlockSpec((r,), lambda i: (i,)),
        out_shape=jax.ShapeDtypeStruct((N,), jnp.float32),
    )(op, dp, b1, w2)


def _tc3_body(q_ref, d_ref, b2_ref, o_ref):
    q = q_ref[0] + q_ref[1]
    d = d_ref[0] + d_ref[1]
    o_ref[...] = q / (d + 1e-16) + b2_ref[0, 0]


def _tc3(q, d2, b2):
    r = 1000
    return pl.pallas_call(
        _tc3_body,
        grid=(N // r,),
        in_specs=[
            pl.BlockSpec((NC, r), lambda i: (0, i)),
            pl.BlockSpec((NC, r), lambda i: (0, i)),
            pl.BlockSpec((1, 1), lambda i: (0, 0)),
        ],
        out_specs=pl.BlockSpec((r,), lambda i: (i,)),
        out_shape=jax.ShapeDtypeStruct((N,), jnp.float32),
    )(q, d2, b2)


@functools.partial(
    pl.kernel,
    out_type=[
        jax.ShapeDtypeStruct((NC, N, H1), jnp.float32),
        jax.ShapeDtypeStruct((NC, N), jnp.float32),
    ],
    mesh=_mesh,
    scratch_types=[
        pltpu.VMEM((NCHUNK, CHUNK), jnp.int32),
        pltpu.VMEM((NCHUNK, CHUNK), jnp.int32),
        pltpu.VMEM((NSB, SB), jnp.int32),
        pltpu.VMEM((NCHUNK, CHUNK), jnp.float32),
        pltpu.VMEM((NCHUNK, CHUNK), jnp.float32),
        pltpu.VMEM((CHUNK, H1), jnp.float32),
        pltpu.VMEM_SHARED((N, H1), jnp.float32),
        pltpu.VMEM_SHARED((N,), jnp.float32),
        pltpu.SemaphoreType.DMA,
    ],
)
def _sc1(er_hbm, er80_hbm, as_hbm, ad_hbm, h_hbm, z16_hbm, z1_hbm,
         out_hbm, den_hbm, src_v, dst_v, dst80_v, ag_v, dg_v, rows_v,
         acc_sh, den_sh, sem):
    c = lax.axis_index("c")
    s = lax.axis_index("s")
    wid = c * NS + s

    @pl.when(s == 0)
    def _():
        pltpu.sync_copy(z16_hbm, acc_sh)
        pltpu.sync_copy(z1_hbm, den_sh)

    pltpu.sync_copy(er_hbm.at[0, wid], src_v)
    pltpu.sync_copy(er_hbm.at[1, wid], dst_v)
    pltpu.sync_copy(er80_hbm.at[1, wid], dst80_v)

    for k in range(NCHUNK):
        pltpu.async_copy(as_hbm.at[src_v.at[k]], ag_v.at[k], sem)
        pltpu.async_copy(ad_hbm.at[dst_v.at[k]], dg_v.at[k], sem)
    for k in range(NCHUNK):
        pltpu.make_async_copy(as_hbm.at[src_v.at[k]], ag_v.at[k], sem).wait()
        pltpu.make_async_copy(ad_hbm.at[dst_v.at[k]], dg_v.at[k], sem).wait()

    for k in range(NCHUNK):
        def _ex_body(t, _):
            sl = pl.ds(t * 16, 16)
            ag_v[k, sl] = _leaky_exp(ag_v[k, sl] + dg_v[k, sl])
            return 0
        lax.fori_loop(0, CHUNK // 16, _ex_body, 0)

    plsc.subcore_barrier()

    def _den_body(j, _):
        pltpu.sync_copy(
            ag_v.at[j // SBC, pl.ds((j % SBC) * SB, SB)],
            den_sh.at[dst80_v.at[j]],
            add=True,
        )
        return 0
    lax.fori_loop(0, NSB, _den_body, 0)

    for k in range(NCHUNK):
        pltpu.async_copy(h_hbm.at[src_v.at[k]], rows_v, sem).wait()

        def _scale_body(e, _):
            rows_v[e] = rows_v[e] * ag_v[k, e]
            return 0
        lax.fori_loop(0, CHUNK, _scale_body, 0)

        def _sc_body(j, _):
            pltpu.sync_copy(
                rows_v.at[pl.ds(j * SB, SB)],
                acc_sh.at[dst80_v.at[k * SBC + j]],
                add=True,
            )
            return 0
        lax.fori_loop(0, SBC, _sc_body, 0)

    plsc.subcore_barrier()

    rows_per = 624
    off = s * rows_per
    pltpu.sync_copy(acc_sh.at[pl.ds(off, rows_per)],
                    out_hbm.at[c, pl.ds(off, rows_per)])

    @pl.when(s == NS - 1)
    def _():
        tail = NS * rows_per
        pltpu.sync_copy(acc_sh.at[pl.ds(tail, N - tail)],
                        out_hbm.at[c, pl.ds(tail, N - tail)])

    @pl.when(s == 0)
    def _():
        pltpu.sync_copy(den_sh, den_hbm.at[c])


@functools.partial(
    pl.kernel,
    out_type=[
        jax.ShapeDtypeStruct((NC, N), jnp.float32),
        jax.ShapeDtypeStruct((NC, N), jnp.float32),
    ],
    mesh=_mesh,
    scratch_types=[
        pltpu.VMEM((NCHUNK, CHUNK), jnp.int32),
        pltpu.VMEM((NCHUNK, CHUNK), jnp.int32),
        pltpu.VMEM((NSB, SB), jnp.int32),
        pltpu.VMEM((NCHUNK, CHUNK), jnp.float32),
        pltpu.VMEM((NCHUNK, CHUNK), jnp.float32),
        pltpu.VMEM((16,), jnp.float32),
        pltpu.VMEM((16,), jnp.float32),
        pltpu.VMEM_SHARED((N,), jnp.float32),
        pltpu.VMEM_SHARED((N,), jnp.float32),
        pltpu.SemaphoreType.DMA,
    ],
)
def _sc2(er_hbm, er80_hbm, h2_hbm, a2s_hbm, a2d_hbm, z1_hbm,
         out_hbm, den_hbm, src_v, dst_v, dst80_v, hs_v, hd_v, as_v, ad_v,
         acc_sh, den_sh, sem):
    c = lax.axis_index("c")
    s = lax.axis_index("s")
    wid = c * NS + s

    @pl.when(s == 0)
    def _():
        pltpu.sync_copy(z1_hbm, acc_sh)
        pltpu.sync_copy(z1_hbm, den_sh)

    pltpu.sync_copy(er_hbm.at[0, wid], src_v)
    pltpu.sync_copy(er_hbm.at[1, wid], dst_v)
    pltpu.sync_copy(er80_hbm.at[1, wid], dst80_v)
    pltpu.sync_copy(a2s_hbm, as_v)
    pltpu.sync_copy(a2d_hbm, ad_v)

    for k in range(NCHUNK):
        pltpu.async_copy(h2_hbm.at[src_v.at[k]], hs_v.at[k], sem)
        pltpu.async_copy(h2_hbm.at[dst_v.at[k]], hd_v.at[k], sem)
    for k in range(NCHUNK):
        pltpu.make_async_copy(h2_hbm.at[src_v.at[k]], hs_v.at[k], sem).wait()
        pltpu.make_async_copy(h2_hbm.at[dst_v.at[k]], hd_v.at[k], sem).wait()

    att_s = as_v[...]
    att_d = ad_v[...]
    for k in range(NCHUNK):
        def _ex_body(t, _):
            sl = pl.ds(t * 16, 16)
            hs = hs_v[k, sl]
            ex = _leaky_exp(att_s * hs + att_d * hd_v[k, sl])
            hs_v[k, sl] = ex * hs
            hd_v[k, sl] = ex
            return 0
        lax.fori_loop(0, CHUNK // 16, _ex_body, 0)

    plsc.subcore_barrier()

    def _scat_body(j, _):
        idx = dst80_v.at[j]
        sl = pl.ds((j % SBC) * SB, SB)
        pltpu.sync_copy(hs_v.at[j // SBC, sl], acc_sh.at[idx], add=True)
        pltpu.sync_copy(hd_v.at[j // SBC, sl], den_sh.at[idx], add=True)
        return 0
    lax.fori_loop(0, NSB, _scat_body, 0)

    plsc.subcore_barrier()

    @pl.when(s == 0)
    def _():
        pltpu.sync_copy(acc_sh, out_hbm.at[c])

    @pl.when(s == 1)
    def _():
        pltpu.sync_copy(den_sh, den_hbm.at[c])


def kernel(x, edge_index, W1_src, W1_dst, att1_src, att1_dst, b1,
           W2, att2_src, att2_dst, b2):
    er = edge_index.reshape(2, NW, NCHUNK, CHUNK)
    er80 = edge_index.reshape(2, NW, NSB, SB)
    vsd = jnp.stack([W1_src @ att1_src, W1_dst @ att1_dst], axis=1)
    z16 = jnp.zeros((N, H1), jnp.float32)
    z1 = jnp.zeros((N,), jnp.float32)

    h_src, oa = _tc1(x, W1_src, vsd)
    a_s = oa[:, 0]
    a_d = oa[:, 1]

    out1p, den1p = _sc1(er, er80, a_s, a_d, h_src, z16, z1)

    h2s = _tc2(out1p, den1p, b1.reshape(1, H1), W2)

    a2s = jnp.full((16,), att2_src[0], jnp.float32)
    a2d = jnp.full((16,), att2_dst[0], jnp.float32)
    out2p, den2p = _sc2(er, er80, h2s, a2s, a2d, z1)

    out = _tc3(out2p, den2p, b2.reshape(1, 1))
    return out.reshape(N, 1)


# speedup vs baseline: 75.9738x; 75.9738x over previous
import functools

import jax
import jax.numpy as jnp
from jax import lax
from jax.experimental import pallas as pl
from jax.experimental.pallas import tpu as pltpu
from jax.experimental.pallas import tpu_sc as plsc

N = 10000
E = 320000
D = 128
H1 = 16

NC = 2
NS = 16
NW = NC * NS
EW = E // NW
NCHUNK = 5
CHUNK = EW // NCHUNK
SB = 80
NSB = EW // SB
SBC = CHUNK // SB

_mesh = plsc.VectorSubcoreMesh(
    core_axis_name="c", subcore_axis_name="s", num_cores=NC, num_subcores=NS
)


def _leaky_exp(v):
    return jnp.exp(jnp.where(v >= 0.0, v, 0.2 * v))


def _tc1_body(x_ref, w_ref, v_ref, h_ref, oa_ref):
    xb = x_ref[...]
    h_ref[...] = jnp.dot(xb, w_ref[...], preferred_element_type=jnp.float32)
    oa_ref[...] = jnp.dot(xb, v_ref[...], preferred_element_type=jnp.float32)


def _tc1(x, w1s, vsd):
    return pl.pallas_call(
        _tc1_body,
        out_shape=[
            jax.ShapeDtypeStruct((N, H1), jnp.float32),
            jax.ShapeDtypeStruct((N, 2), jnp.float32),
        ],
    )(x, w1s, vsd)


def _tc2_body(op_ref, dp_ref, b1_ref, w2_ref, h2_ref):
    p = op_ref[0] + op_ref[1]
    d = dp_ref[0] + dp_ref[1]
    h1 = jnp.maximum(p / (d[:, None] + 1e-16) + b1_ref[...], 0.0)
    h2 = jnp.dot(h1, w2_ref[...], preferred_element_type=jnp.float32)
    h2_ref[...] = h2[:, 0]


def _tc2(op, dp, b1, w2):
    return pl.pallas_call(
        _tc2_body,
        out_shape=jax.ShapeDtypeStruct((N,), jnp.float32),
    )(op, dp, b1, w2)


def _tc3_body(q_ref, d_ref, b2_ref, o_ref):
    q = q_ref[0] + q_ref[1]
    d = d_ref[0] + d_ref[1]
    o_ref[...] = q / (d + 1e-16) + b2_ref[0, 0]


def _tc3(q, d2, b2):
    return pl.pallas_call(
        _tc3_body,
        out_shape=jax.ShapeDtypeStruct((N,), jnp.float32),
    )(q, d2, b2)


@functools.partial(
    pl.kernel,
    out_type=[
        jax.ShapeDtypeStruct((NC, N, H1), jnp.float32),
        jax.ShapeDtypeStruct((NC, N), jnp.float32),
    ],
    mesh=_mesh,
    compiler_params=pltpu.CompilerParams(use_tc_tiling_on_sc=False),
    scratch_types=[
        pltpu.VMEM((NCHUNK, CHUNK), jnp.int32),
        pltpu.VMEM((NCHUNK, CHUNK), jnp.int32),
        pltpu.VMEM((NCHUNK, CHUNK), jnp.float32),
        pltpu.VMEM((NCHUNK, CHUNK), jnp.float32),
        pltpu.VMEM((CHUNK, H1), jnp.float32),
        pltpu.VMEM_SHARED((N, H1), jnp.float32),
        pltpu.VMEM_SHARED((N,), jnp.float32),
        pltpu.SemaphoreType.DMA,
    ],
)
def _sc1(er_hbm, as_hbm, ad_hbm, h_hbm, z16_hbm, z1_hbm,
         out_hbm, den_hbm, src_v, dst_v, ag_v, dg_v, rows_v,
         acc_sh, den_sh, sem):
    c = lax.axis_index("c")
    s = lax.axis_index("s")
    wid = c * NS + s

    @pl.when(s == 0)
    def _():
        pltpu.sync_copy(z16_hbm, acc_sh)
        pltpu.sync_copy(z1_hbm, den_sh)

    pltpu.sync_copy(er_hbm.at[0, wid], src_v)
    pltpu.sync_copy(er_hbm.at[1, wid], dst_v)

    for k in range(NCHUNK):
        pltpu.async_copy(as_hbm.at[src_v.at[k]], ag_v.at[k], sem)
        pltpu.async_copy(ad_hbm.at[dst_v.at[k]], dg_v.at[k], sem)
    for k in range(NCHUNK):
        pltpu.make_async_copy(as_hbm.at[src_v.at[k]], ag_v.at[k], sem).wait()
        pltpu.make_async_copy(ad_hbm.at[dst_v.at[k]], dg_v.at[k], sem).wait()

    for k in range(NCHUNK):
        def _ex_body(t, _):
            sl = pl.ds(t * 16, 16)
            ag_v[k, sl] = _leaky_exp(ag_v[k, sl] + dg_v[k, sl])
            return 0
        lax.fori_loop(0, CHUNK // 16, _ex_body, 0)

    plsc.subcore_barrier()

    def _den_body(j, _):
        pltpu.sync_copy(
            ag_v.at[j // SBC, pl.ds((j % SBC) * SB, SB)],
            den_sh.at[dst_v.at[j // SBC, pl.ds((j % SBC) * SB, SB)]],
            add=True,
        )
        return 0
    lax.fori_loop(0, NSB, _den_body, 0)

    for k in range(NCHUNK):
        pltpu.async_copy(h_hbm.at[src_v.at[k]], rows_v, sem).wait()

        def _scale_body(m, _):
            wv = ag_v[k, pl.ds(m * 16, 16)]
            for j in range(16):
                rows_v[m * 16 + j] = rows_v[m * 16 + j] * wv[j]
            return 0
        lax.fori_loop(0, CHUNK // 16, _scale_body, 0)

        def _sc_body(j, _):
            pltpu.sync_copy(
                rows_v.at[pl.ds(j * SB, SB)],
                acc_sh.at[dst_v.at[k, pl.ds(j * SB, SB)]],
                add=True,
            )
            return 0
        lax.fori_loop(0, SBC, _sc_body, 0)

    plsc.subcore_barrier()

    rows_per = 624
    off = s * rows_per
    pltpu.sync_copy(acc_sh.at[pl.ds(off, rows_per)],
                    out_hbm.at[c, pl.ds(off, rows_per)])

    @pl.when(s == NS - 1)
    def _():
        tail = NS * rows_per
        pltpu.sync_copy(acc_sh.at[pl.ds(tail, N - tail)],
                        out_hbm.at[c, pl.ds(tail, N - tail)])

    @pl.when(s == 0)
    def _():
        pltpu.sync_copy(den_sh, den_hbm.at[c])


@functools.partial(
    pl.kernel,
    out_type=[
        jax.ShapeDtypeStruct((NC, N), jnp.float32),
        jax.ShapeDtypeStruct((NC, N), jnp.float32),
    ],
    mesh=_mesh,
    compiler_params=pltpu.CompilerParams(use_tc_tiling_on_sc=False),
    scratch_types=[
        pltpu.VMEM((NCHUNK, CHUNK), jnp.int32),
        pltpu.VMEM((NCHUNK, CHUNK), jnp.int32),
        pltpu.VMEM((NCHUNK, CHUNK), jnp.float32),
        pltpu.VMEM((NCHUNK, CHUNK), jnp.float32),
        pltpu.VMEM((16,), jnp.float32),
        pltpu.VMEM((16,), jnp.float32),
        pltpu.VMEM_SHARED((N,), jnp.float32),
        pltpu.VMEM_SHARED((N,), jnp.float32),
        pltpu.SemaphoreType.DMA,
    ],
)
def _sc2(er_hbm, h2_hbm, a2s_hbm, a2d_hbm, z1_hbm,
         out_hbm, den_hbm, src_v, dst_v, hs_v, hd_v, as_v, ad_v,
         acc_sh, den_sh, sem):
    c = lax.axis_index("c")
    s = lax.axis_index("s")
    wid = c * NS + s

    @pl.when(s == 0)
    def _():
        pltpu.sync_copy(z1_hbm, acc_sh)
        pltpu.sync_copy(z1_hbm, den_sh)

    pltpu.sync_copy(er_hbm.at[0, wid], src_v)
    pltpu.sync_copy(er_hbm.at[1, wid], dst_v)
    pltpu.sync_copy(a2s_hbm, as_v)
    pltpu.sync_copy(a2d_hbm, ad_v)

    for k in range(NCHUNK):
        pltpu.async_copy(h2_hbm.at[src_v.at[k]], hs_v.at[k], sem)
        pltpu.async_copy(h2_hbm.at[dst_v.at[k]], hd_v.at[k], sem)
    for k in range(NCHUNK):
        pltpu.make_async_copy(h2_hbm.at[src_v.at[k]], hs_v.at[k], sem).wait()
        pltpu.make_async_copy(h2_hbm.at[dst_v.at[k]], hd_v.at[k], sem).wait()

    att_s = as_v[...]
    att_d = ad_v[...]
    for k in range(NCHUNK):
        def _ex_body(t, _):
            sl = pl.ds(t * 16, 16)
            hs = hs_v[k, sl]
            ex = _leaky_exp(att_s * hs + att_d * hd_v[k, sl])
            hs_v[k, sl] = ex * hs
            hd_v[k, sl] = ex
            return 0
        lax.fori_loop(0, CHUNK // 16, _ex_body, 0)

    plsc.subcore_barrier()

    def _scat_body(j, _):
        sl = pl.ds((j % SBC) * SB, SB)
        idx = dst_v.at[j // SBC, sl]
        pltpu.sync_copy(hs_v.at[j // SBC, sl], acc_sh.at[idx], add=True)
        pltpu.sync_copy(hd_v.at[j // SBC, sl], den_sh.at[idx], add=True)
        return 0
    lax.fori_loop(0, NSB, _scat_body, 0)

    plsc.subcore_barrier()

    @pl.when(s == 0)
    def _():
        pltpu.sync_copy(acc_sh, out_hbm.at[c])

    @pl.when(s == 1)
    def _():
        pltpu.sync_copy(den_sh, den_hbm.at[c])


def kernel(x, edge_index, W1_src, W1_dst, att1_src, att1_dst, b1,
           W2, att2_src, att2_dst, b2):
    er = edge_index.reshape(2, NW, NCHUNK, CHUNK)
    vsd = jnp.stack([W1_src @ att1_src, W1_dst @ att1_dst], axis=1)
    z16 = jnp.zeros((N, H1), jnp.float32)
    z1 = jnp.zeros((N,), jnp.float32)

    h_src, oa = _tc1(x, W1_src, vsd)
    a_s = oa[:, 0]
    a_d = oa[:, 1]

    out1p, den1p = _sc1(er, a_s, a_d, h_src, z16, z1)

    h2s = _tc2(out1p, den1p, b1.reshape(1, H1), W2)

    a2s = jnp.full((16,), att2_src[0], jnp.float32)
    a2d = jnp.full((16,), att2_dst[0], jnp.float32)
    out2p, den2p = _sc2(er, h2s, a2s, a2d, z1)

    out = _tc3(out2p, den2p, b2.reshape(1, 1))
    return out.reshape(N, 1)


# speedup vs baseline: 88.7379x; 1.1680x over previous
import functools

import jax
import jax.numpy as jnp
from jax import lax
from jax.experimental import pallas as pl
from jax.experimental.pallas import tpu as pltpu
from jax.experimental.pallas import tpu_sc as plsc

N = 10000
E = 320000
D = 128
H1 = 16

NC = 2
NS = 16
NW = NC * NS
EW = E // NW
NCHUNK = 5
CHUNK = EW // NCHUNK
SB = 80
NSB = EW // SB
SBC = CHUNK // SB

_mesh = plsc.VectorSubcoreMesh(
    core_axis_name="c", subcore_axis_name="s", num_cores=NC, num_subcores=NS
)


def _leaky_exp(v):
    return jnp.exp(jnp.where(v >= 0.0, v, 0.2 * v))


def _tc1_body(x_ref, w_ref, v_ref, h_ref, oa_ref):
    xb = x_ref[...]
    h_ref[...] = jnp.dot(xb, w_ref[...], preferred_element_type=jnp.float32)
    oa_ref[...] = jnp.dot(xb, v_ref[...], preferred_element_type=jnp.float32)


def _tc1(x, w1s, vsd):
    return pl.pallas_call(
        _tc1_body,
        out_shape=[
            jax.ShapeDtypeStruct((N, H1), jnp.float32),
            jax.ShapeDtypeStruct((N, 2), jnp.float32),
        ],
    )(x, w1s, vsd)


def _tc2_body(op_ref, dp_ref, b1_ref, w2_ref, h2_ref):
    p = op_ref[0] + op_ref[1]
    d = dp_ref[0] + dp_ref[1]
    h1 = jnp.maximum(p / (d[:, None] + 1e-16) + b1_ref[...], 0.0)
    h2 = jnp.dot(h1, w2_ref[...], preferred_element_type=jnp.float32)
    h2_ref[...] = h2[:, 0]


def _tc2(op, dp, b1, w2):
    return pl.pallas_call(
        _tc2_body,
        out_shape=jax.ShapeDtypeStruct((N,), jnp.float32),
    )(op, dp, b1, w2)


def _tc3_body(q_ref, d_ref, b2_ref, o_ref):
    q = q_ref[0] + q_ref[1]
    d = d_ref[0] + d_ref[1]
    o_ref[...] = q / (d + 1e-16) + b2_ref[0, 0]


def _tc3(q, d2, b2):
    return pl.pallas_call(
        _tc3_body,
        out_shape=jax.ShapeDtypeStruct((N,), jnp.float32),
    )(q, d2, b2)


@functools.partial(
    pl.kernel,
    out_type=[
        jax.ShapeDtypeStruct((NC, N, H1), jnp.float32),
        jax.ShapeDtypeStruct((NC, N), jnp.float32),
    ],
    mesh=_mesh,
    compiler_params=pltpu.CompilerParams(use_tc_tiling_on_sc=False),
    scratch_types=[
        pltpu.VMEM((NCHUNK, CHUNK), jnp.int32),
        pltpu.VMEM((NCHUNK, CHUNK), jnp.int32),
        pltpu.VMEM((NCHUNK, CHUNK), jnp.float32),
        pltpu.VMEM((NCHUNK, CHUNK), jnp.float32),
        pltpu.VMEM((2, CHUNK, H1), jnp.float32),
        pltpu.VMEM_SHARED((N, H1), jnp.float32),
        pltpu.VMEM_SHARED((N,), jnp.float32),
        pltpu.SemaphoreType.DMA,
        pltpu.SemaphoreType.DMA,
        pltpu.SemaphoreType.DMA,
        pltpu.SemaphoreType.DMA,
    ],
)
def _sc1(er_hbm, as_hbm, ad_hbm, h_hbm, z16_hbm, z1_hbm,
         out_hbm, den_hbm, src_v, dst_v, ag_v, dg_v, rows_v,
         acc_sh, den_sh, sem, rsem, dsem, ssem):
    c = lax.axis_index("c")
    s = lax.axis_index("s")
    wid = c * NS + s

    @pl.when(s == 0)
    def _():
        pltpu.sync_copy(z16_hbm, acc_sh)
        pltpu.sync_copy(z1_hbm, den_sh)

    pltpu.async_copy(er_hbm.at[0, wid], src_v, sem)
    pltpu.async_copy(er_hbm.at[1, wid], dst_v, sem)
    pltpu.make_async_copy(er_hbm.at[0, wid], src_v, sem).wait()

    for k in range(2):
        pltpu.async_copy(h_hbm.at[src_v.at[k]], rows_v.at[k], rsem)

    pltpu.make_async_copy(er_hbm.at[1, wid], dst_v, sem).wait()

    for k in range(NCHUNK):
        pltpu.async_copy(as_hbm.at[src_v.at[k]], ag_v.at[k], sem)
        pltpu.async_copy(ad_hbm.at[dst_v.at[k]], dg_v.at[k], sem)
    for k in range(NCHUNK):
        pltpu.make_async_copy(as_hbm.at[src_v.at[k]], ag_v.at[k], sem).wait()
        pltpu.make_async_copy(ad_hbm.at[dst_v.at[k]], dg_v.at[k], sem).wait()

    for k in range(NCHUNK):
        def _ex_body(t, _):
            sl = pl.ds(t * 16, 16)
            ag_v[k, sl] = _leaky_exp(ag_v[k, sl] + dg_v[k, sl])
            return 0
        lax.fori_loop(0, CHUNK // 16, _ex_body, 0, unroll=4)

    plsc.subcore_barrier()

    def _den_issue(j, _):
        sl = pl.ds((j % SBC) * SB, SB)
        pltpu.async_copy(ag_v.at[j // SBC, sl],
                         den_sh.at[dst_v.at[j // SBC, sl]], dsem, add=True)
        return 0
    lax.fori_loop(0, NSB, _den_issue, 0)

    for k in range(NCHUNK):
        b = k % 2
        pltpu.make_async_copy(h_hbm.at[src_v.at[k]], rows_v.at[b], rsem).wait()

        def _scale_body(m, _):
            wv = ag_v[k, pl.ds(m * 16, 16)]
            for j in range(16):
                rows_v[b, m * 16 + j] = rows_v[b, m * 16 + j] * wv[j]
            return 0
        lax.fori_loop(0, CHUNK // 16, _scale_body, 0)

        def _sc_issue(j, _):
            sl = pl.ds(j * SB, SB)
            pltpu.async_copy(rows_v.at[b, sl],
                             acc_sh.at[dst_v.at[k, sl]], ssem, add=True)
            return 0
        lax.fori_loop(0, SBC, _sc_issue, 0)

        if k + 2 < NCHUNK:
            def _sc_drain(j, _):
                sl = pl.ds(j * SB, SB)
                pltpu.make_async_copy(rows_v.at[b, sl],
                                      acc_sh.at[dst_v.at[k, sl]],
                                      ssem).wait()
                return 0
            lax.fori_loop(0, SBC, _sc_drain, 0)
            pltpu.async_copy(h_hbm.at[src_v.at[k + 2]], rows_v.at[b], rsem)

    for k in (NCHUNK - 2, NCHUNK - 1):
        b = k % 2
        def _sc_drain2(j, _):
            sl = pl.ds(j * SB, SB)
            pltpu.make_async_copy(rows_v.at[b, sl],
                                  acc_sh.at[dst_v.at[k, sl]], ssem).wait()
            return 0
        lax.fori_loop(0, SBC, _sc_drain2, 0)

    def _den_drain(j, _):
        sl = pl.ds((j % SBC) * SB, SB)
        pltpu.make_async_copy(ag_v.at[j // SBC, sl],
                              den_sh.at[dst_v.at[j // SBC, sl]], dsem).wait()
        return 0
    lax.fori_loop(0, NSB, _den_drain, 0)

    plsc.subcore_barrier()

    rows_per = 624
    off = s * rows_per
    pltpu.sync_copy(acc_sh.at[pl.ds(off, rows_per)],
                    out_hbm.at[c, pl.ds(off, rows_per)])

    @pl.when(s == NS - 1)
    def _():
        tail = NS * rows_per
        pltpu.sync_copy(acc_sh.at[pl.ds(tail, N - tail)],
                        out_hbm.at[c, pl.ds(tail, N - tail)])

    @pl.when(s == 0)
    def _():
        pltpu.sync_copy(den_sh, den_hbm.at[c])


@functools.partial(
    pl.kernel,
    out_type=[
        jax.ShapeDtypeStruct((NC, N), jnp.float32),
        jax.ShapeDtypeStruct((NC, N), jnp.float32),
    ],
    mesh=_mesh,
    compiler_params=pltpu.CompilerParams(use_tc_tiling_on_sc=False),
    scratch_types=[
        pltpu.VMEM((NCHUNK, CHUNK), jnp.int32),
        pltpu.VMEM((NCHUNK, CHUNK), jnp.int32),
        pltpu.VMEM((NCHUNK, CHUNK), jnp.float32),
        pltpu.VMEM((NCHUNK, CHUNK), jnp.float32),
        pltpu.VMEM((16,), jnp.float32),
        pltpu.VMEM((16,), jnp.float32),
        pltpu.VMEM_SHARED((N,), jnp.float32),
        pltpu.VMEM_SHARED((N,), jnp.float32),
        pltpu.SemaphoreType.DMA,
        pltpu.SemaphoreType.DMA,
    ],
)
def _sc2(er_hbm, h2_hbm, a2s_hbm, a2d_hbm, z1_hbm,
         out_hbm, den_hbm, src_v, dst_v, hs_v, hd_v, as_v, ad_v,
         acc_sh, den_sh, sem, ssem):
    c = lax.axis_index("c")
    s = lax.axis_index("s")
    wid = c * NS + s

    @pl.when(s == 0)
    def _():
        pltpu.sync_copy(z1_hbm, acc_sh)
        pltpu.sync_copy(z1_hbm, den_sh)

    pltpu.sync_copy(er_hbm.at[0, wid], src_v)
    pltpu.sync_copy(er_hbm.at[1, wid], dst_v)
    pltpu.sync_copy(a2s_hbm, as_v)
    pltpu.sync_copy(a2d_hbm, ad_v)

    for k in range(NCHUNK):
        pltpu.async_copy(h2_hbm.at[src_v.at[k]], hs_v.at[k], sem)
        pltpu.async_copy(h2_hbm.at[dst_v.at[k]], hd_v.at[k], sem)
    for k in range(NCHUNK):
        pltpu.make_async_copy(h2_hbm.at[src_v.at[k]], hs_v.at[k], sem).wait()
        pltpu.make_async_copy(h2_hbm.at[dst_v.at[k]], hd_v.at[k], sem).wait()

    att_s = as_v[...]
    att_d = ad_v[...]
    for k in range(NCHUNK):
        def _ex_body(t, _):
            sl = pl.ds(t * 16, 16)
            hs = hs_v[k, sl]
            ex = _leaky_exp(att_s * hs + att_d * hd_v[k, sl])
            hs_v[k, sl] = ex * hs
            hd_v[k, sl] = ex
            return 0
        lax.fori_loop(0, CHUNK // 16, _ex_body, 0, unroll=4)

    plsc.subcore_barrier()

    def _scat_issue(j, _):
        sl = pl.ds((j % SBC) * SB, SB)
        idx = dst_v.at[j // SBC, sl]
        pltpu.async_copy(hs_v.at[j // SBC, sl], acc_sh.at[idx], ssem, add=True)
        pltpu.async_copy(hd_v.at[j // SBC, sl], den_sh.at[idx], ssem, add=True)
        return 0
    lax.fori_loop(0, NSB, _scat_issue, 0)

    def _scat_drain(j, _):
        sl = pl.ds((j % SBC) * SB, SB)
        idx = dst_v.at[j // SBC, sl]
        pltpu.make_async_copy(hs_v.at[j // SBC, sl], acc_sh.at[idx],
                              ssem).wait()
        pltpu.make_async_copy(hd_v.at[j // SBC, sl], den_sh.at[idx],
                              ssem).wait()
        return 0
    lax.fori_loop(0, NSB, _scat_drain, 0)

    plsc.subcore_barrier()

    @pl.when(s == 0)
    def _():
        pltpu.sync_copy(acc_sh, out_hbm.at[c])

    @pl.when(s == 1)
    def _():
        pltpu.sync_copy(den_sh, den_hbm.at[c])


def kernel(x, edge_index, W1_src, W1_dst, att1_src, att1_dst, b1,
           W2, att2_src, att2_dst, b2):
    er = edge_index.reshape(2, NW, NCHUNK, CHUNK)
    vsd = jnp.stack([W1_src @ att1_src, W1_dst @ att1_dst], axis=1)
    z16 = jnp.zeros((N, H1), jnp.float32)
    z1 = jnp.zeros((N,), jnp.float32)

    h_src, oa = _tc1(x, W1_src, vsd)
    a_s = oa[:, 0]
    a_d = oa[:, 1]

    out1p, den1p = _sc1(er, a_s, a_d, h_src, z16, z1)

    h2s = _tc2(out1p, den1p, b1.reshape(1, H1), W2)

    a2s = jnp.full((16,), att2_src[0], jnp.float32)
    a2d = jnp.full((16,), att2_dst[0], jnp.float32)
    out2p, den2p = _sc2(er, h2s, a2s, a2d, z1)

    out = _tc3(out2p, den2p, b2.reshape(1, 1))
    return out.reshape(N, 1)


# speedup vs baseline: 90.0359x; 1.0146x over previous
import functools

import jax
import jax.numpy as jnp
from jax import lax
from jax.experimental import pallas as pl
from jax.experimental.pallas import tpu as pltpu
from jax.experimental.pallas import tpu_sc as plsc

N = 10000
E = 320000
D = 128
H1 = 16

NC = 2
NS = 16
NW = NC * NS
EW = E // NW
NCHUNK = 5
CHUNK = EW // NCHUNK
SB = 80
NSB = EW // SB
SBC = CHUNK // SB

_mesh = plsc.VectorSubcoreMesh(
    core_axis_name="c", subcore_axis_name="s", num_cores=NC, num_subcores=NS
)


def _leaky_exp(v):
    return jnp.exp(jnp.where(v >= 0.0, v, 0.2 * v))


def _tc1_body(x_ref, w_ref, v_ref, h_ref, oa_ref):
    xb = x_ref[...]
    h_ref[...] = jnp.dot(xb, w_ref[...], preferred_element_type=jnp.float32)
    oa_ref[...] = jnp.dot(xb, v_ref[...], preferred_element_type=jnp.float32)


def _tc1(x, w1s, vsd):
    return pl.pallas_call(
        _tc1_body,
        out_shape=[
            jax.ShapeDtypeStruct((N, H1), jnp.float32),
            jax.ShapeDtypeStruct((N, 2), jnp.float32),
        ],
    )(x, w1s, vsd)


def _tc2_body(op_ref, dp_ref, b1_ref, w2_ref, h2_ref):
    p = op_ref[0] + op_ref[1]
    d = dp_ref[0] + dp_ref[1]
    h1 = jnp.maximum(p / (d[:, None] + 1e-16) + b1_ref[...], 0.0)
    h2 = jnp.dot(h1, w2_ref[...], preferred_element_type=jnp.float32)
    h2_ref[...] = h2[:, 0]


def _tc2(op, dp, b1, w2):
    return pl.pallas_call(
        _tc2_body,
        out_shape=jax.ShapeDtypeStruct((N,), jnp.float32),
    )(op, dp, b1, w2)


def _tc3_body(q_ref, d_ref, b2_ref, o_ref):
    q = q_ref[0] + q_ref[1]
    d = d_ref[0] + d_ref[1]
    o_ref[...] = q / (d + 1e-16) + b2_ref[0, 0]


def _tc3(q, d2, b2):
    return pl.pallas_call(
        _tc3_body,
        out_shape=jax.ShapeDtypeStruct((N,), jnp.float32),
    )(q, d2, b2)


@functools.partial(
    pl.kernel,
    out_type=[
        jax.ShapeDtypeStruct((NC, N, H1), jnp.float32),
        jax.ShapeDtypeStruct((NC, N), jnp.float32),
    ],
    mesh=_mesh,
    compiler_params=pltpu.CompilerParams(use_tc_tiling_on_sc=False),
    scratch_types=[
        pltpu.VMEM((NCHUNK, CHUNK), jnp.int32),
        pltpu.VMEM((NCHUNK, CHUNK), jnp.int32),
        pltpu.VMEM((NCHUNK, CHUNK), jnp.float32),
        pltpu.VMEM((NCHUNK, CHUNK), jnp.float32),
        pltpu.VMEM((2, CHUNK, H1), jnp.float32),
        pltpu.VMEM_SHARED((N, H1), jnp.float32),
        pltpu.VMEM_SHARED((N,), jnp.float32),
        pltpu.SemaphoreType.DMA,
        pltpu.SemaphoreType.DMA,
        pltpu.SemaphoreType.DMA,
        pltpu.SemaphoreType.DMA,
    ],
)
def _sc1(er_hbm, as_hbm, ad_hbm, h_hbm, z16_hbm, z1_hbm,
         out_hbm, den_hbm, src_v, dst_v, ag_v, dg_v, rows_v,
         acc_sh, den_sh, sem, rsem, dsem, ssem):
    c = lax.axis_index("c")
    s = lax.axis_index("s")
    wid = c * NS + s

    @pl.when(s == 0)
    def _():
        pltpu.sync_copy(z16_hbm, acc_sh)
        pltpu.sync_copy(z1_hbm, den_sh)

    pltpu.async_copy(er_hbm.at[0, wid], src_v, sem)
    pltpu.async_copy(er_hbm.at[1, wid], dst_v, sem)
    pltpu.make_async_copy(er_hbm.at[0, wid], src_v, sem).wait()

    for k in range(2):
        pltpu.async_copy(h_hbm.at[src_v.at[k]], rows_v.at[k], rsem)

    pltpu.make_async_copy(er_hbm.at[1, wid], dst_v, sem).wait()

    for k in range(NCHUNK):
        pltpu.async_copy(as_hbm.at[src_v.at[k]], ag_v.at[k], sem)
        pltpu.async_copy(ad_hbm.at[dst_v.at[k]], dg_v.at[k], sem)
    for k in range(NCHUNK):
        pltpu.make_async_copy(as_hbm.at[src_v.at[k]], ag_v.at[k], sem).wait()
        pltpu.make_async_copy(ad_hbm.at[dst_v.at[k]], dg_v.at[k], sem).wait()

    for k in range(NCHUNK):
        def _ex_body(t, _):
            sl = pl.ds(t * 16, 16)
            ag_v[k, sl] = _leaky_exp(ag_v[k, sl] + dg_v[k, sl])
            return 0
        lax.fori_loop(0, CHUNK // 16, _ex_body, 0, unroll=4)

    plsc.subcore_barrier()

    for k in range(NCHUNK):
        pltpu.async_copy(ag_v.at[k], den_sh.at[dst_v.at[k]], dsem, add=True)

    for k in range(NCHUNK):
        b = k % 2
        pltpu.make_async_copy(h_hbm.at[src_v.at[k]], rows_v.at[b], rsem).wait()

        def _scale_body(m, _):
            wv = ag_v[k, pl.ds(m * 16, 16)]
            for j in range(16):
                rows_v[b, m * 16 + j] = rows_v[b, m * 16 + j] * wv[j]
            return 0
        lax.fori_loop(0, CHUNK // 16, _scale_body, 0)

        pltpu.async_copy(rows_v.at[b], acc_sh.at[dst_v.at[k]], ssem,
                         add=True)

        if k + 2 < NCHUNK:
            pltpu.make_async_copy(rows_v.at[b], acc_sh.at[dst_v.at[k]],
                                  ssem).wait()
            pltpu.async_copy(h_hbm.at[src_v.at[k + 2]], rows_v.at[b], rsem)

    for k in (NCHUNK - 2, NCHUNK - 1):
        b = k % 2
        pltpu.make_async_copy(rows_v.at[b], acc_sh.at[dst_v.at[k]],
                              ssem).wait()
    for k in range(NCHUNK):
        pltpu.make_async_copy(ag_v.at[k], den_sh.at[dst_v.at[k]],
                              dsem).wait()

    plsc.subcore_barrier()

    rows_per = 624
    off = s * rows_per
    pltpu.sync_copy(acc_sh.at[pl.ds(off, rows_per)],
                    out_hbm.at[c, pl.ds(off, rows_per)])

    @pl.when(s == NS - 1)
    def _():
        tail = NS * rows_per
        pltpu.sync_copy(acc_sh.at[pl.ds(tail, N - tail)],
                        out_hbm.at[c, pl.ds(tail, N - tail)])

    @pl.when(s == 0)
    def _():
        pltpu.sync_copy(den_sh, den_hbm.at[c])


@functools.partial(
    pl.kernel,
    out_type=[
        jax.ShapeDtypeStruct((NC, N), jnp.float32),
        jax.ShapeDtypeStruct((NC, N), jnp.float32),
    ],
    mesh=_mesh,
    compiler_params=pltpu.CompilerParams(use_tc_tiling_on_sc=False),
    scratch_types=[
        pltpu.VMEM((NCHUNK, CHUNK), jnp.int32),
        pltpu.VMEM((NCHUNK, CHUNK), jnp.int32),
        pltpu.VMEM((NCHUNK, CHUNK), jnp.float32),
        pltpu.VMEM((NCHUNK, CHUNK), jnp.float32),
        pltpu.VMEM((16,), jnp.float32),
        pltpu.VMEM((16,), jnp.float32),
        pltpu.VMEM_SHARED((N,), jnp.float32),
        pltpu.VMEM_SHARED((N,), jnp.float32),
        pltpu.SemaphoreType.DMA,
        pltpu.SemaphoreType.DMA,
    ],
)
def _sc2(er_hbm, h2_hbm, a2s_hbm, a2d_hbm, z1_hbm,
         out_hbm, den_hbm, src_v, dst_v, hs_v, hd_v, as_v, ad_v,
         acc_sh, den_sh, sem, ssem):
    c = lax.axis_index("c")
    s = lax.axis_index("s")
    wid = c * NS + s

    @pl.when(s == 0)
    def _():
        pltpu.sync_copy(z1_hbm, acc_sh)
        pltpu.sync_copy(z1_hbm, den_sh)

    pltpu.async_copy(er_hbm.at[0, wid], src_v, sem)
    pltpu.async_copy(er_hbm.at[1, wid], dst_v, sem)
    pltpu.async_copy(a2s_hbm, as_v, sem)
    pltpu.async_copy(a2d_hbm, ad_v, sem)
    pltpu.make_async_copy(er_hbm.at[0, wid], src_v, sem).wait()
    pltpu.make_async_copy(er_hbm.at[1, wid], dst_v, sem).wait()
    pltpu.make_async_copy(a2s_hbm, as_v, sem).wait()
    pltpu.make_async_copy(a2d_hbm, ad_v, sem).wait()

    for k in range(NCHUNK):
        pltpu.async_copy(h2_hbm.at[src_v.at[k]], hs_v.at[k], sem)
        pltpu.async_copy(h2_hbm.at[dst_v.at[k]], hd_v.at[k], sem)
    for k in range(NCHUNK):
        pltpu.make_async_copy(h2_hbm.at[src_v.at[k]], hs_v.at[k], sem).wait()
        pltpu.make_async_copy(h2_hbm.at[dst_v.at[k]], hd_v.at[k], sem).wait()

    att_s = as_v[...]
    att_d = ad_v[...]
    for k in range(NCHUNK):
        def _ex_body(t, _):
            sl = pl.ds(t * 16, 16)
            hs = hs_v[k, sl]
            ex = _leaky_exp(att_s * hs + att_d * hd_v[k, sl])
            hs_v[k, sl] = ex * hs
            hd_v[k, sl] = ex
            return 0
        lax.fori_loop(0, CHUNK // 16, _ex_body, 0, unroll=4)

    plsc.subcore_barrier()

    for k in range(NCHUNK):
        pltpu.async_copy(hs_v.at[k], acc_sh.at[dst_v.at[k]], ssem, add=True)
        pltpu.async_copy(hd_v.at[k], den_sh.at[dst_v.at[k]], ssem, add=True)
    for k in range(NCHUNK):
        pltpu.make_async_copy(hs_v.at[k], acc_sh.at[dst_v.at[k]],
                              ssem).wait()
        pltpu.make_async_copy(hd_v.at[k], den_sh.at[dst_v.at[k]],
                              ssem).wait()

    plsc.subcore_barrier()

    @pl.when(s == 0)
    def _():
        pltpu.sync_copy(acc_sh, out_hbm.at[c])

    @pl.when(s == 1)
    def _():
        pltpu.sync_copy(den_sh, den_hbm.at[c])


def kernel(x, edge_index, W1_src, W1_dst, att1_src, att1_dst, b1,
           W2, att2_src, att2_dst, b2):
    er = edge_index.reshape(2, NW, NCHUNK, CHUNK)
    vsd = jnp.stack([W1_src @ att1_src, W1_dst @ att1_dst], axis=1)
    z16 = jnp.zeros((N, H1), jnp.float32)
    z1 = jnp.zeros((N,), jnp.float32)

    h_src, oa = _tc1(x, W1_src, vsd)
    a_s = oa[:, 0]
    a_d = oa[:, 1]

    out1p, den1p = _sc1(er, a_s, a_d, h_src, z16, z1)

    h2s = _tc2(out1p, den1p, b1.reshape(1, H1), W2)

    a2s = jnp.full((16,), att2_src[0], jnp.float32)
    a2d = jnp.full((16,), att2_dst[0], jnp.float32)
    out2p, den2p = _sc2(er, h2s, a2s, a2d, z1)

    out = _tc3(out2p, den2p, b2.reshape(1, 1))
    return out.reshape(N, 1)


# speedup vs baseline: 90.0379x; 1.0000x over previous
import functools

import jax
import jax.numpy as jnp
from jax import lax
from jax.experimental import pallas as pl
from jax.experimental.pallas import tpu as pltpu
from jax.experimental.pallas import tpu_sc as plsc

N = 10000
E = 320000
D = 128
H1 = 16

NC = 2
NS = 16
NW = NC * NS
EW = E // NW
NCHUNK = 5
CHUNK = EW // NCHUNK
SB = 80
NSB = EW // SB
SBC = CHUNK // SB

_mesh = plsc.VectorSubcoreMesh(
    core_axis_name="c", subcore_axis_name="s", num_cores=NC, num_subcores=NS
)


def _leaky_exp(v):
    return jnp.exp(jnp.where(v >= 0.0, v, 0.2 * v))


def _tc1_body(x_ref, w_ref, v_ref, h_ref, oa_ref):
    xb = x_ref[...]
    h_ref[...] = jnp.dot(xb, w_ref[...], preferred_element_type=jnp.float32)
    oa_ref[...] = jnp.dot(xb, v_ref[...], preferred_element_type=jnp.float32)


def _tc1(x, w1s, vsd):
    return pl.pallas_call(
        _tc1_body,
        out_shape=[
            jax.ShapeDtypeStruct((N, H1), jnp.float32),
            jax.ShapeDtypeStruct((N, 2), jnp.float32),
        ],
    )(x, w1s, vsd)


def _tc2_body(op_ref, dp_ref, b1_ref, w2_ref, h2_ref):
    p = op_ref[0] + op_ref[1]
    d = dp_ref[0] + dp_ref[1]
    h1 = jnp.maximum(p / (d[:, None] + 1e-16) + b1_ref[...], 0.0)
    h2 = jnp.dot(h1, w2_ref[...], preferred_element_type=jnp.float32)
    h2_ref[...] = h2[:, 0]


def _tc2(op, dp, b1, w2):
    return pl.pallas_call(
        _tc2_body,
        out_shape=jax.ShapeDtypeStruct((N,), jnp.float32),
    )(op, dp, b1, w2)


def _tc3_body(q_ref, d_ref, b2_ref, o_ref):
    q = q_ref[0] + q_ref[1]
    d = d_ref[0] + d_ref[1]
    o_ref[...] = q / (d + 1e-16) + b2_ref[0, 0]


def _tc3(q, d2, b2):
    return pl.pallas_call(
        _tc3_body,
        out_shape=jax.ShapeDtypeStruct((N,), jnp.float32),
    )(q, d2, b2)


@functools.partial(
    pl.kernel,
    out_type=[
        jax.ShapeDtypeStruct((NC, N, H1), jnp.float32),
        jax.ShapeDtypeStruct((NC, N), jnp.float32),
    ],
    mesh=_mesh,
    compiler_params=pltpu.CompilerParams(use_tc_tiling_on_sc=False),
    scratch_types=[
        pltpu.VMEM((NCHUNK, CHUNK), jnp.int32),
        pltpu.VMEM((NCHUNK, CHUNK), jnp.int32),
        pltpu.VMEM((NCHUNK, CHUNK), jnp.float32),
        pltpu.VMEM((NCHUNK, CHUNK), jnp.float32),
        pltpu.VMEM((2, CHUNK, H1), jnp.float32),
        pltpu.VMEM_SHARED((N, H1), jnp.float32),
        pltpu.VMEM_SHARED((N,), jnp.float32),
        pltpu.SemaphoreType.DMA,
        pltpu.SemaphoreType.DMA,
        pltpu.SemaphoreType.DMA,
        pltpu.SemaphoreType.DMA,
    ],
)
def _sc1(er_hbm, as_hbm, ad_hbm, h_hbm, z16_hbm, z1_hbm,
         out_hbm, den_hbm, src_v, dst_v, ag_v, dg_v, rows_v,
         acc_sh, den_sh, sem, rsem, dsem, ssem):
    c = lax.axis_index("c")
    s = lax.axis_index("s")
    wid = c * NS + s

    @pl.when(s == 0)
    def _():
        pltpu.sync_copy(z16_hbm, acc_sh)
        pltpu.sync_copy(z1_hbm, den_sh)

    pltpu.async_copy(er_hbm.at[0, wid], src_v, sem)
    pltpu.async_copy(er_hbm.at[1, wid], dst_v, sem)
    pltpu.make_async_copy(er_hbm.at[0, wid], src_v, sem).wait()

    for k in range(2):
        pltpu.async_copy(h_hbm.at[src_v.at[k]], rows_v.at[k], rsem)

    pltpu.make_async_copy(er_hbm.at[1, wid], dst_v, sem).wait()

    with jax.named_scope("sc1_logit_gather"):
        for k in range(NCHUNK):
            pltpu.async_copy(as_hbm.at[src_v.at[k]], ag_v.at[k], sem)
            pltpu.async_copy(ad_hbm.at[dst_v.at[k]], dg_v.at[k], sem)
        for k in range(NCHUNK):
            pltpu.make_async_copy(as_hbm.at[src_v.at[k]], ag_v.at[k], sem).wait()
            pltpu.make_async_copy(ad_hbm.at[dst_v.at[k]], dg_v.at[k], sem).wait()

    with jax.named_scope("sc1_ex"):
        for k in range(NCHUNK):
            def _ex_body(t, _):
                sl = pl.ds(t * 16, 16)
                ag_v[k, sl] = _leaky_exp(ag_v[k, sl] + dg_v[k, sl])
                return 0
            lax.fori_loop(0, CHUNK // 16, _ex_body, 0, unroll=4)

    with jax.named_scope("sc1_bar1"):
        plsc.subcore_barrier()

    for k in range(NCHUNK):
        pltpu.async_copy(ag_v.at[k], den_sh.at[dst_v.at[k]], dsem, add=True)

    for k in range(NCHUNK):
        b = k % 2
        with jax.named_scope("sc1_rowwait"):
            pltpu.make_async_copy(h_hbm.at[src_v.at[k]], rows_v.at[b],
                                  rsem).wait()

        with jax.named_scope("sc1_scale"):
            def _scale_body(m, _):
                wv = ag_v[k, pl.ds(m * 16, 16)]
                for j in range(16):
                    rows_v[b, m * 16 + j] = rows_v[b, m * 16 + j] * wv[j]
                return 0
            lax.fori_loop(0, CHUNK // 16, _scale_body, 0)

        pltpu.async_copy(rows_v.at[b], acc_sh.at[dst_v.at[k]], ssem,
                         add=True)

        if k + 2 < NCHUNK:
            pltpu.make_async_copy(rows_v.at[b], acc_sh.at[dst_v.at[k]],
                                  ssem).wait()
            pltpu.async_copy(h_hbm.at[src_v.at[k + 2]], rows_v.at[b], rsem)

    with jax.named_scope("sc1_scatdrain"):
        for k in (NCHUNK - 2, NCHUNK - 1):
            b = k % 2
            pltpu.make_async_copy(rows_v.at[b], acc_sh.at[dst_v.at[k]],
                                  ssem).wait()
        for k in range(NCHUNK):
            pltpu.make_async_copy(ag_v.at[k], den_sh.at[dst_v.at[k]],
                                  dsem).wait()

    with jax.named_scope("sc1_bar2"):
        plsc.subcore_barrier()

    rows_per = 624
    off = s * rows_per
    pltpu.sync_copy(acc_sh.at[pl.ds(off, rows_per)],
                    out_hbm.at[c, pl.ds(off, rows_per)])

    @pl.when(s == NS - 1)
    def _():
        tail = NS * rows_per
        pltpu.sync_copy(acc_sh.at[pl.ds(tail, N - tail)],
                        out_hbm.at[c, pl.ds(tail, N - tail)])

    @pl.when(s == 0)
    def _():
        pltpu.sync_copy(den_sh, den_hbm.at[c])


@functools.partial(
    pl.kernel,
    out_type=[
        jax.ShapeDtypeStruct((NC, N), jnp.float32),
        jax.ShapeDtypeStruct((NC, N), jnp.float32),
    ],
    mesh=_mesh,
    compiler_params=pltpu.CompilerParams(use_tc_tiling_on_sc=False),
    scratch_types=[
        pltpu.VMEM((NCHUNK, CHUNK), jnp.int32),
        pltpu.VMEM((NCHUNK, CHUNK), jnp.int32),
        pltpu.VMEM((NCHUNK, CHUNK), jnp.float32),
        pltpu.VMEM((NCHUNK, CHUNK), jnp.float32),
        pltpu.VMEM((16,), jnp.float32),
        pltpu.VMEM((16,), jnp.float32),
        pltpu.VMEM_SHARED((N,), jnp.float32),
        pltpu.VMEM_SHARED((N,), jnp.float32),
        pltpu.SemaphoreType.DMA,
        pltpu.SemaphoreType.DMA,
    ],
)
def _sc2(er_hbm, h2_hbm, a2s_hbm, a2d_hbm, z1_hbm,
         out_hbm, den_hbm, src_v, dst_v, hs_v, hd_v, as_v, ad_v,
         acc_sh, den_sh, sem, ssem):
    c = lax.axis_index("c")
    s = lax.axis_index("s")
    wid = c * NS + s

    @pl.when(s == 0)
    def _():
        pltpu.sync_copy(z1_hbm, acc_sh)
        pltpu.sync_copy(z1_hbm, den_sh)

    pltpu.async_copy(er_hbm.at[0, wid], src_v, sem)
    pltpu.async_copy(er_hbm.at[1, wid], dst_v, sem)
    pltpu.async_copy(a2s_hbm, as_v, sem)
    pltpu.async_copy(a2d_hbm, ad_v, sem)
    pltpu.make_async_copy(er_hbm.at[0, wid], src_v, sem).wait()
    pltpu.make_async_copy(er_hbm.at[1, wid], dst_v, sem).wait()
    pltpu.make_async_copy(a2s_hbm, as_v, sem).wait()
    pltpu.make_async_copy(a2d_hbm, ad_v, sem).wait()

    with jax.named_scope("sc2_gather"):
        for k in range(NCHUNK):
            pltpu.async_copy(h2_hbm.at[src_v.at[k]], hs_v.at[k], sem)
            pltpu.async_copy(h2_hbm.at[dst_v.at[k]], hd_v.at[k], sem)
        for k in range(NCHUNK):
            pltpu.make_async_copy(h2_hbm.at[src_v.at[k]], hs_v.at[k],
                                  sem).wait()
            pltpu.make_async_copy(h2_hbm.at[dst_v.at[k]], hd_v.at[k],
                                  sem).wait()

    att_s = as_v[...]
    att_d = ad_v[...]
    with jax.named_scope("sc2_ex"):
        for k in range(NCHUNK):
            def _ex_body(t, _):
                sl = pl.ds(t * 16, 16)
                hs = hs_v[k, sl]
                ex = _leaky_exp(att_s * hs + att_d * hd_v[k, sl])
                hs_v[k, sl] = ex * hs
                hd_v[k, sl] = ex
                return 0
            lax.fori_loop(0, CHUNK // 16, _ex_body, 0, unroll=4)

    with jax.named_scope("sc2_bar1"):
        plsc.subcore_barrier()

    with jax.named_scope("sc2_scatter"):
        for k in range(NCHUNK):
            pltpu.async_copy(hs_v.at[k], acc_sh.at[dst_v.at[k]], ssem,
                             add=True)
            pltpu.async_copy(hd_v.at[k], den_sh.at[dst_v.at[k]], ssem,
                             add=True)
        for k in range(NCHUNK):
            pltpu.make_async_copy(hs_v.at[k], acc_sh.at[dst_v.at[k]],
                                  ssem).wait()
            pltpu.make_async_copy(hd_v.at[k], den_sh.at[dst_v.at[k]],
                                  ssem).wait()

    with jax.named_scope("sc2_bar2"):
        plsc.subcore_barrier()

    @pl.when(s == 0)
    def _():
        pltpu.sync_copy(acc_sh, out_hbm.at[c])

    @pl.when(s == 1)
    def _():
        pltpu.sync_copy(den_sh, den_hbm.at[c])


def kernel(x, edge_index, W1_src, W1_dst, att1_src, att1_dst, b1,
           W2, att2_src, att2_dst, b2):
    er = edge_index.reshape(2, NW, NCHUNK, CHUNK)
    vsd = jnp.stack([W1_src @ att1_src, W1_dst @ att1_dst], axis=1)
    z16 = jnp.zeros((N, H1), jnp.float32)
    z1 = jnp.zeros((N,), jnp.float32)

    h_src, oa = _tc1(x, W1_src, vsd)
    a_s = oa[:, 0]
    a_d = oa[:, 1]

    out1p, den1p = _sc1(er, a_s, a_d, h_src, z16, z1)

    h2s = _tc2(out1p, den1p, b1.reshape(1, H1), W2)

    a2s = jnp.full((16,), att2_src[0], jnp.float32)
    a2d = jnp.full((16,), att2_dst[0], jnp.float32)
    out2p, den2p = _sc2(er, h2s, a2s, a2d, z1)

    out = _tc3(out2p, den2p, b2.reshape(1, 1))
    return out.reshape(N, 1)


# speedup vs baseline: 166.1648x; 1.8455x over previous
import functools

import jax
import jax.numpy as jnp
from jax import lax
from jax.experimental import pallas as pl
from jax.experimental.pallas import tpu as pltpu
from jax.experimental.pallas import tpu_sc as plsc

N = 10000
E = 320000
D = 128
H1 = 16

NC = 2
NS = 16
NW = NC * NS
EW = E // NW
NCHUNK = 5
CHUNK = EW // NCHUNK
SB = 80
NSB = EW // SB
SBC = CHUNK // SB

_mesh = plsc.VectorSubcoreMesh(
    core_axis_name="c", subcore_axis_name="s", num_cores=NC, num_subcores=NS
)


def _leaky_exp(v):
    return jnp.exp(jnp.where(v >= 0.0, v, 0.2 * v))


def _tc1_body(x_ref, w_ref, v_ref, h_ref, oa_ref):
    xb = x_ref[...]
    h_ref[...] = jnp.dot(xb, w_ref[...], preferred_element_type=jnp.float32)
    oa_ref[...] = jnp.dot(xb, v_ref[...], preferred_element_type=jnp.float32)


def _tc1(x, w1s, vsd):
    return pl.pallas_call(
        _tc1_body,
        out_shape=[
            jax.ShapeDtypeStruct((N, H1), jnp.float32),
            jax.ShapeDtypeStruct((N, 2), jnp.float32),
        ],
    )(x, w1s, vsd)


def _tc2_body(op_ref, dp_ref, b1_ref, w2_ref, h2_ref):
    p = op_ref[0] + op_ref[1]
    d = dp_ref[0] + dp_ref[1]
    h1 = jnp.maximum(p / (d[:, None] + 1e-16) + b1_ref[...], 0.0)
    h2 = jnp.dot(h1, w2_ref[...], preferred_element_type=jnp.float32)
    h2_ref[...] = h2[:, 0]


def _tc2(op, dp, b1, w2):
    return pl.pallas_call(
        _tc2_body,
        out_shape=jax.ShapeDtypeStruct((N,), jnp.float32),
    )(op, dp, b1, w2)


def _tc3_body(q_ref, d_ref, b2_ref, o_ref):
    q = q_ref[0] + q_ref[1]
    d = d_ref[0] + d_ref[1]
    o_ref[...] = q / (d + 1e-16) + b2_ref[0, 0]


def _tc3(q, d2, b2):
    return pl.pallas_call(
        _tc3_body,
        out_shape=jax.ShapeDtypeStruct((N,), jnp.float32),
    )(q, d2, b2)


@functools.partial(
    pl.kernel,
    out_type=[
        jax.ShapeDtypeStruct((NC, N, H1), jnp.float32),
        jax.ShapeDtypeStruct((NC, N), jnp.float32),
    ],
    mesh=_mesh,
    compiler_params=pltpu.CompilerParams(use_tc_tiling_on_sc=False, needs_layout_passes=False),
    scratch_types=[
        pltpu.VMEM((NCHUNK, CHUNK), jnp.int32),
        pltpu.VMEM((NCHUNK, CHUNK), jnp.int32),
        pltpu.VMEM((N,), jnp.float32),
        pltpu.VMEM((N,), jnp.float32),
        pltpu.VMEM((NCHUNK, CHUNK), jnp.float32),
        pltpu.VMEM((2, CHUNK, H1), jnp.float32),
        pltpu.VMEM_SHARED((N, H1), jnp.float32),
        pltpu.VMEM_SHARED((N,), jnp.float32),
        pltpu.SemaphoreType.DMA,
        pltpu.SemaphoreType.DMA,
        pltpu.SemaphoreType.DMA,
        pltpu.SemaphoreType.DMA,
    ],
)
def _sc1(er_hbm, as_hbm, ad_hbm, h_hbm, z16_hbm, z1_hbm,
         out_hbm, den_hbm, src_v, dst_v, as_t, ad_t, ag_v, rows_v,
         acc_sh, den_sh, sem, rsem, dsem, ssem):
    c = lax.axis_index("c")
    s = lax.axis_index("s")
    wid = c * NS + s

    @pl.when(s == 0)
    def _():
        pltpu.sync_copy(z16_hbm, acc_sh)
        pltpu.sync_copy(z1_hbm, den_sh)

    pltpu.async_copy(er_hbm.at[0, wid], src_v, sem)
    pltpu.async_copy(er_hbm.at[1, wid], dst_v, sem)
    pltpu.make_async_copy(er_hbm.at[0, wid], src_v, sem).wait()

    for k in range(2):
        pltpu.async_copy(h_hbm.at[src_v.at[k]], rows_v.at[k], rsem)

    pltpu.make_async_copy(er_hbm.at[1, wid], dst_v, sem).wait()

    with jax.named_scope("sc1_logit_gather"):
        pltpu.async_copy(as_hbm, as_t, sem)
        pltpu.async_copy(ad_hbm, ad_t, sem)
        pltpu.make_async_copy(as_hbm, as_t, sem).wait()
        pltpu.make_async_copy(ad_hbm, ad_t, sem).wait()

    with jax.named_scope("sc1_ex"):
        for k in range(NCHUNK):
            def _ex_body(t, _):
                sl = pl.ds(t * 16, 16)
                av = plsc.load_gather(as_t, [src_v[k, sl]])
                dv = plsc.load_gather(ad_t, [dst_v[k, sl]])
                ag_v[k, sl] = _leaky_exp(av + dv)
                return 0
            lax.fori_loop(0, CHUNK // 16, _ex_body, 0, unroll=4)

    with jax.named_scope("sc1_bar1"):
        plsc.subcore_barrier()

    for k in range(NCHUNK):
        pltpu.async_copy(ag_v.at[k], den_sh.at[dst_v.at[k]], dsem, add=True)

    for k in range(NCHUNK):
        b = k % 2
        with jax.named_scope("sc1_rowwait"):
            pltpu.make_async_copy(h_hbm.at[src_v.at[k]], rows_v.at[b],
                                  rsem).wait()

        with jax.named_scope("sc1_scale"):
            def _scale_body(m, _):
                wv = ag_v[k, pl.ds(m * 16, 16)]
                for j in range(16):
                    rows_v[b, m * 16 + j] = rows_v[b, m * 16 + j] * wv[j]
                return 0
            lax.fori_loop(0, CHUNK // 16, _scale_body, 0)

        pltpu.async_copy(rows_v.at[b], acc_sh.at[dst_v.at[k]], ssem,
                         add=True)

        if k + 2 < NCHUNK:
            pltpu.make_async_copy(rows_v.at[b], acc_sh.at[dst_v.at[k]],
                                  ssem).wait()
            pltpu.async_copy(h_hbm.at[src_v.at[k + 2]], rows_v.at[b], rsem)

    with jax.named_scope("sc1_scatdrain"):
        for k in (NCHUNK - 2, NCHUNK - 1):
            b = k % 2
            pltpu.make_async_copy(rows_v.at[b], acc_sh.at[dst_v.at[k]],
                                  ssem).wait()
        for k in range(NCHUNK):
            pltpu.make_async_copy(ag_v.at[k], den_sh.at[dst_v.at[k]],
                                  dsem).wait()

    with jax.named_scope("sc1_bar2"):
        plsc.subcore_barrier()

    rows_per = 624
    off = s * rows_per
    pltpu.sync_copy(acc_sh.at[pl.ds(off, rows_per)],
                    out_hbm.at[c, pl.ds(off, rows_per)])

    @pl.when(s == NS - 1)
    def _():
        tail = NS * rows_per
        pltpu.sync_copy(acc_sh.at[pl.ds(tail, N - tail)],
                        out_hbm.at[c, pl.ds(tail, N - tail)])

    @pl.when(s == 0)
    def _():
        pltpu.sync_copy(den_sh, den_hbm.at[c])


@functools.partial(
    pl.kernel,
    out_type=[
        jax.ShapeDtypeStruct((NC, N), jnp.float32),
        jax.ShapeDtypeStruct((NC, N), jnp.float32),
    ],
    mesh=_mesh,
    compiler_params=pltpu.CompilerParams(use_tc_tiling_on_sc=False, needs_layout_passes=False),
    scratch_types=[
        pltpu.VMEM((NCHUNK, CHUNK), jnp.int32),
        pltpu.VMEM((NCHUNK, CHUNK), jnp.int32),
        pltpu.VMEM((N,), jnp.float32),
        pltpu.VMEM((NCHUNK, CHUNK), jnp.float32),
        pltpu.VMEM((NCHUNK, CHUNK), jnp.float32),
        pltpu.VMEM((16,), jnp.float32),
        pltpu.VMEM((16,), jnp.float32),
        pltpu.VMEM_SHARED((N,), jnp.float32),
        pltpu.VMEM_SHARED((N,), jnp.float32),
        pltpu.SemaphoreType.DMA,
        pltpu.SemaphoreType.DMA,
    ],
)
def _sc2(er_hbm, h2_hbm, a2s_hbm, a2d_hbm, z1_hbm,
         out_hbm, den_hbm, src_v, dst_v, h2_t, hs_v, hd_v, as_v, ad_v,
         acc_sh, den_sh, sem, ssem):
    c = lax.axis_index("c")
    s = lax.axis_index("s")
    wid = c * NS + s

    @pl.when(s == 0)
    def _():
        pltpu.sync_copy(z1_hbm, acc_sh)
        pltpu.sync_copy(z1_hbm, den_sh)

    pltpu.async_copy(er_hbm.at[0, wid], src_v, sem)
    pltpu.async_copy(er_hbm.at[1, wid], dst_v, sem)
    pltpu.async_copy(a2s_hbm, as_v, sem)
    pltpu.async_copy(a2d_hbm, ad_v, sem)
    with jax.named_scope("sc2_gather"):
        pltpu.async_copy(h2_hbm, h2_t, sem)
        pltpu.make_async_copy(er_hbm.at[0, wid], src_v, sem).wait()
        pltpu.make_async_copy(er_hbm.at[1, wid], dst_v, sem).wait()
        pltpu.make_async_copy(a2s_hbm, as_v, sem).wait()
        pltpu.make_async_copy(a2d_hbm, ad_v, sem).wait()
        pltpu.make_async_copy(h2_hbm, h2_t, sem).wait()

    att_s = as_v[...]
    att_d = ad_v[...]
    with jax.named_scope("sc2_ex"):
        for k in range(NCHUNK):
            def _ex_body(t, _):
                sl = pl.ds(t * 16, 16)
                hs = plsc.load_gather(h2_t, [src_v[k, sl]])
                hd = plsc.load_gather(h2_t, [dst_v[k, sl]])
                ex = _leaky_exp(att_s * hs + att_d * hd)
                hs_v[k, sl] = ex * hs
                hd_v[k, sl] = ex
                return 0
            lax.fori_loop(0, CHUNK // 16, _ex_body, 0, unroll=4)

    with jax.named_scope("sc2_bar1"):
        plsc.subcore_barrier()

    with jax.named_scope("sc2_scatter"):
        for k in range(NCHUNK):
            pltpu.async_copy(hs_v.at[k], acc_sh.at[dst_v.at[k]], ssem,
                             add=True)
            pltpu.async_copy(hd_v.at[k], den_sh.at[dst_v.at[k]], ssem,
                             add=True)
        for k in range(NCHUNK):
            pltpu.make_async_copy(hs_v.at[k], acc_sh.at[dst_v.at[k]],
                                  ssem).wait()
            pltpu.make_async_copy(hd_v.at[k], den_sh.at[dst_v.at[k]],
                                  ssem).wait()

    with jax.named_scope("sc2_bar2"):
        plsc.subcore_barrier()

    @pl.when(s == 0)
    def _():
        pltpu.sync_copy(acc_sh, out_hbm.at[c])

    @pl.when(s == 1)
    def _():
        pltpu.sync_copy(den_sh, den_hbm.at[c])


def kernel(x, edge_index, W1_src, W1_dst, att1_src, att1_dst, b1,
           W2, att2_src, att2_dst, b2):
    er = edge_index.reshape(2, NW, NCHUNK, CHUNK)
    vsd = jnp.stack([W1_src @ att1_src, W1_dst @ att1_dst], axis=1)
    z16 = jnp.zeros((N, H1), jnp.float32)
    z1 = jnp.zeros((N,), jnp.float32)

    h_src, oa = _tc1(x, W1_src, vsd)
    a_s = oa[:, 0]
    a_d = oa[:, 1]

    out1p, den1p = _sc1(er, a_s, a_d, h_src, z16, z1)

    h2s = _tc2(out1p, den1p, b1.reshape(1, H1), W2)

    a2s = jnp.full((16,), att2_src[0], jnp.float32)
    a2d = jnp.full((16,), att2_dst[0], jnp.float32)
    out2p, den2p = _sc2(er, h2s, a2s, a2d, z1)

    out = _tc3(out2p, den2p, b2.reshape(1, 1))
    return out.reshape(N, 1)
